# baseline (speedup 1.0000x reference)
import functools

import jax
import jax.numpy as jnp
from jax import lax
from jax.experimental import pallas as pl
from jax.experimental.pallas import tpu as pltpu
from jax.experimental.pallas import tpu_sc as plsc

_B = 16384
_D = 32
_F = 32
_H = 128
_NC = 2
_NS = 16
_NW = _NC * _NS
_BPW = _B // _NW

_BLK = 1024
_NBLK = _B // _BLK


def _gather_body(uids, iids, ue, uf, ie, it,
                 oue, ouf, oie, oit,
                 uidv, iidv, sem):
    wid = lax.axis_index("s") * _NC + lax.axis_index("c")
    base = wid * _BPW
    pltpu.sync_copy(uids.at[pl.ds(base, _BPW)], uidv)
    pltpu.sync_copy(iids.at[pl.ds(base, _BPW)], iidv)

    @pl.loop(0, _BPW // 16)
    def _row_group(t):
        off = t * 16
        uvec = uidv[pl.ds(off, 16)]
        ivec = iidv[pl.ds(off, 16)]
        for l in range(16):
            us = uvec[l]
            vs = ivec[l]
            k = base + t * 16 + l
            pltpu.async_copy(ue.at[pl.ds(us, 1)], oue.at[pl.ds(k, 1)], sem)
            pltpu.async_copy(uf.at[pl.ds(us, 1)], ouf.at[pl.ds(k, 1)], sem)
            pltpu.async_copy(ie.at[pl.ds(vs, 1)], oie.at[pl.ds(k, 1)], sem)
            pltpu.async_copy(it.at[pl.ds(vs, 1)], oit.at[pl.ds(k, 1)], sem)

    pltpu.make_async_copy(ue.at[pl.ds(0, _BPW)],
                          oue.at[pl.ds(base, _BPW)], sem).wait()
    pltpu.make_async_copy(uf.at[pl.ds(0, _BPW)],
                          ouf.at[pl.ds(base, _BPW)], sem).wait()
    pltpu.make_async_copy(ie.at[pl.ds(0, _BPW)],
                          oie.at[pl.ds(base, _BPW)], sem).wait()
    pltpu.make_async_copy(it.at[pl.ds(0, _BPW)],
                          oit.at[pl.ds(base, _BPW)], sem).wait()


@functools.lru_cache(maxsize=1)
def _make_gather():
    return pl.kernel(
        _gather_body,
        out_type=(
            jax.ShapeDtypeStruct((_B, _D), jnp.float32),
            jax.ShapeDtypeStruct((_B, _F), jnp.float32),
            jax.ShapeDtypeStruct((_B, _D), jnp.float32),
            jax.ShapeDtypeStruct((_B, _F), jnp.float32),
        ),
        mesh=plsc.VectorSubcoreMesh(core_axis_name="c", subcore_axis_name="s",
                                    num_cores=_NC, num_subcores=_NS),
        scratch_types=[
            pltpu.VMEM((_BPW,), jnp.int32),
            pltpu.VMEM((_BPW,), jnp.int32),
            pltpu.SemaphoreType.DMA,
        ],
    )


def _tower(e, f, w1a, w1b, b1, g, beta, w2, b2):
    h = (jnp.dot(e, w1a, preferred_element_type=jnp.float32)
         + jnp.dot(f, w1b, preferred_element_type=jnp.float32) + b1)
    h = jnp.maximum(h, 0.0)
    m = jnp.mean(h, axis=-1, keepdims=True)
    v = jnp.mean(jnp.square(h - m), axis=-1, keepdims=True)
    h = (h - m) / jnp.sqrt(v + 1e-5) * g + beta
    z = jnp.dot(h, w2, preferred_element_type=jnp.float32) + b2
    n = jnp.sqrt(jnp.sum(z * z, axis=-1, keepdims=True))
    return z / jnp.maximum(n, 1e-12)


def _tower_body(ue, uf, ie, it,
                uw1a, uw1b, ub1, ug, ubeta, uw2, ub2,
                iw1a, iw1b, ib1, ig, ibeta, iw2, ib2, out):
    uv = _tower(ue[...], uf[...], uw1a[...], uw1b[...], ub1[...], ug[...],
                ubeta[...], uw2[...], ub2[...])
    iv = _tower(ie[...], it[...], iw1a[...], iw1b[...], ib1[...], ig[...],
                ibeta[...], iw2[...], ib2[...])
    out[...] = jnp.sum(uv * iv, axis=-1, keepdims=True)


def _row_spec(cols):
    return pl.BlockSpec((_BLK, cols), lambda i: (i, 0))


def _full_spec(r, c):
    return pl.BlockSpec((r, c), lambda i: (0, 0))


_towers = pl.pallas_call(
    _tower_body,
    grid=(_NBLK,),
    in_specs=[
        _row_spec(_D), _row_spec(_F), _row_spec(_D), _row_spec(_F),
        _full_spec(_D, _H), _full_spec(_F, _H), _full_spec(1, _H),
        _full_spec(1, _H), _full_spec(1, _H), _full_spec(_H, _D),
        _full_spec(1, _D),
        _full_spec(_D, _H), _full_spec(_F, _H), _full_spec(1, _H),
        _full_spec(1, _H), _full_spec(1, _H), _full_spec(_H, _D),
        _full_spec(1, _D),
    ],
    out_specs=pl.BlockSpec((_BLK, 1), lambda i: (i, 0)),
    out_shape=jax.ShapeDtypeStruct((_B, 1), jnp.float32),
)


def kernel(user_ids, item_ids, user_feats, item_feats, user_emb, item_emb,
           u_W1, u_b1, u_g, u_beta, u_W2, u_b2,
           i_W1, i_b1, i_g, i_beta, i_W2, i_b2):
    uids = user_ids.astype(jnp.int32)
    iids = item_ids.astype(jnp.int32)
    gue, guf, gie, git = _make_gather()(uids, iids, user_emb, user_feats,
                                        item_emb, item_feats)
    scores = _towers(
        gue, guf, gie, git,
        u_W1[:_D], u_W1[_D:], u_b1.reshape(1, _H), u_g.reshape(1, _H),
        u_beta.reshape(1, _H), u_W2, u_b2.reshape(1, _D),
        i_W1[:_D], i_W1[_D:], i_b1.reshape(1, _H), i_g.reshape(1, _H),
        i_beta.reshape(1, _H), i_W2, i_b2.reshape(1, _D),
    )
    return scores.reshape(_B)


# speedup vs baseline: 2.3920x; 2.3920x over previous
import functools

import jax
import jax.numpy as jnp
from jax import lax
from jax.experimental import pallas as pl
from jax.experimental.pallas import tpu as pltpu
from jax.experimental.pallas import tpu_sc as plsc

_B = 16384
_D = 32
_F = 32
_H = 128
_NC = 2
_NS = 16
_NW = _NC * _NS
_BPW = _B // _NW

_BLK = 1024
_NBLK = _B // _BLK


_CH = 128
_NCH = _BPW // _CH


def _gather_body(uids, iids, ue, uf, ie, it,
                 oue, ouf, oie, oit,
                 uidv, iidv, bue, buf, bie, bit, sem):
    wid = lax.axis_index("s") * _NC + lax.axis_index("c")
    base = wid * _BPW
    pltpu.sync_copy(uids.at[pl.ds(base, _BPW)], uidv)
    pltpu.sync_copy(iids.at[pl.ds(base, _BPW)], iidv)
    for c in range(_NCH):
        @pl.loop(0, _CH // 16)
        def _row_group(t):
            off = c * _CH + t * 16
            uvec = uidv[pl.ds(off, 16)]
            ivec = iidv[pl.ds(off, 16)]
            for l in range(16):
                us = uvec[l]
                vs = ivec[l]
                k = t * 16 + l
                pltpu.async_copy(ue.at[pl.ds(us, 1)], bue.at[pl.ds(k, 1)], sem)
                pltpu.async_copy(uf.at[pl.ds(us, 1)], buf.at[pl.ds(k, 1)], sem)
                pltpu.async_copy(ie.at[pl.ds(vs, 1)], bie.at[pl.ds(k, 1)], sem)
                pltpu.async_copy(it.at[pl.ds(vs, 1)], bit.at[pl.ds(k, 1)], sem)
        pltpu.make_async_copy(ue.at[pl.ds(0, _CH)], bue, sem).wait()
        pltpu.make_async_copy(uf.at[pl.ds(0, _CH)], buf, sem).wait()
        pltpu.make_async_copy(ie.at[pl.ds(0, _CH)], bie, sem).wait()
        pltpu.make_async_copy(it.at[pl.ds(0, _CH)], bit, sem).wait()
        cb = base + c * _CH
        pltpu.sync_copy(bue, oue.at[pl.ds(cb, _CH)])
        pltpu.sync_copy(buf, ouf.at[pl.ds(cb, _CH)])
        pltpu.sync_copy(bie, oie.at[pl.ds(cb, _CH)])
        pltpu.sync_copy(bit, oit.at[pl.ds(cb, _CH)])


@functools.lru_cache(maxsize=1)
def _make_gather():
    return pl.kernel(
        _gather_body,
        out_type=(
            jax.ShapeDtypeStruct((_B, _D), jnp.float32),
            jax.ShapeDtypeStruct((_B, _F), jnp.float32),
            jax.ShapeDtypeStruct((_B, _D), jnp.float32),
            jax.ShapeDtypeStruct((_B, _F), jnp.float32),
        ),
        mesh=plsc.VectorSubcoreMesh(core_axis_name="c", subcore_axis_name="s",
                                    num_cores=_NC, num_subcores=_NS),
        scratch_types=[
            pltpu.VMEM((_BPW,), jnp.int32),
            pltpu.VMEM((_BPW,), jnp.int32),
            pltpu.VMEM((_CH, _D), jnp.float32),
            pltpu.VMEM((_CH, _F), jnp.float32),
            pltpu.VMEM((_CH, _D), jnp.float32),
            pltpu.VMEM((_CH, _F), jnp.float32),
            pltpu.SemaphoreType.DMA,
        ],
    )


def _tower(e, f, w1a, w1b, b1, g, beta, w2, b2):
    h = (jnp.dot(e, w1a, preferred_element_type=jnp.float32)
         + jnp.dot(f, w1b, preferred_element_type=jnp.float32) + b1)
    h = jnp.maximum(h, 0.0)
    m = jnp.mean(h, axis=-1, keepdims=True)
    v = jnp.mean(jnp.square(h - m), axis=-1, keepdims=True)
    h = (h - m) / jnp.sqrt(v + 1e-5) * g + beta
    z = jnp.dot(h, w2, preferred_element_type=jnp.float32) + b2
    n = jnp.sqrt(jnp.sum(z * z, axis=-1, keepdims=True))
    return z / jnp.maximum(n, 1e-12)


def _tower_body(ue, uf, ie, it,
                uw1a, uw1b, ub1, ug, ubeta, uw2, ub2,
                iw1a, iw1b, ib1, ig, ibeta, iw2, ib2, out):
    uv = _tower(ue[...], uf[...], uw1a[...], uw1b[...], ub1[...], ug[...],
                ubeta[...], uw2[...], ub2[...])
    iv = _tower(ie[...], it[...], iw1a[...], iw1b[...], ib1[...], ig[...],
                ibeta[...], iw2[...], ib2[...])
    out[...] = jnp.sum(uv * iv, axis=-1, keepdims=True)


def _row_spec(cols):
    return pl.BlockSpec((_BLK, cols), lambda i: (i, 0))


def _full_spec(r, c):
    return pl.BlockSpec((r, c), lambda i: (0, 0))


_towers = pl.pallas_call(
    _tower_body,
    grid=(_NBLK,),
    in_specs=[
        _row_spec(_D), _row_spec(_F), _row_spec(_D), _row_spec(_F),
        _full_spec(_D, _H), _full_spec(_F, _H), _full_spec(1, _H),
        _full_spec(1, _H), _full_spec(1, _H), _full_spec(_H, _D),
        _full_spec(1, _D),
        _full_spec(_D, _H), _full_spec(_F, _H), _full_spec(1, _H),
        _full_spec(1, _H), _full_spec(1, _H), _full_spec(_H, _D),
        _full_spec(1, _D),
    ],
    out_specs=pl.BlockSpec((_BLK, 1), lambda i: (i, 0)),
    out_shape=jax.ShapeDtypeStruct((_B, 1), jnp.float32),
)


def kernel(user_ids, item_ids, user_feats, item_feats, user_emb, item_emb,
           u_W1, u_b1, u_g, u_beta, u_W2, u_b2,
           i_W1, i_b1, i_g, i_beta, i_W2, i_b2):
    uids = user_ids.astype(jnp.int32)
    iids = item_ids.astype(jnp.int32)
    gue, guf, gie, git = _make_gather()(uids, iids, user_emb, user_feats,
                                        item_emb, item_feats)
    scores = _towers(
        gue, guf, gie, git,
        u_W1[:_D], u_W1[_D:], u_b1.reshape(1, _H), u_g.reshape(1, _H),
        u_beta.reshape(1, _H), u_W2, u_b2.reshape(1, _D),
        i_W1[:_D], i_W1[_D:], i_b1.reshape(1, _H), i_g.reshape(1, _H),
        i_beta.reshape(1, _H), i_W2, i_b2.reshape(1, _D),
    )
    return scores.reshape(_B)


# speedup vs baseline: 2.4032x; 1.0047x over previous
import functools

import jax
import jax.numpy as jnp
from jax import lax
from jax.experimental import pallas as pl
from jax.experimental.pallas import tpu as pltpu
from jax.experimental.pallas import tpu_sc as plsc

_B = 16384
_D = 32
_F = 32
_H = 128
_NC = 2
_NS = 16
_NW = _NC * _NS
_BPW = _B // _NW

_BLK = 1024
_NBLK = _B // _BLK


_CH = 128
_NCH = _BPW // _CH


def _gather_body(ids, ta, tb, oa, ob, idv, ba, bb, sem):
    wid = lax.axis_index("s") * _NC + lax.axis_index("c")
    base = wid * _BPW
    pltpu.sync_copy(ids.at[pl.ds(base, _BPW)], idv)
    for c in range(_NCH):
        @pl.loop(0, _CH // 16)
        def _row_group(t):
            off = c * _CH + t * 16
            vec = idv[pl.ds(off, 16)]
            for l in range(16):
                r = vec[l]
                k = t * 16 + l
                pltpu.async_copy(ta.at[pl.ds(r, 1)], ba.at[pl.ds(k, 1)], sem)
                pltpu.async_copy(tb.at[pl.ds(r, 1)], bb.at[pl.ds(k, 1)], sem)
        pltpu.make_async_copy(ta.at[pl.ds(0, _CH)], ba, sem).wait()
        pltpu.make_async_copy(tb.at[pl.ds(0, _CH)], bb, sem).wait()
        cb = base + c * _CH
        pltpu.sync_copy(ba, oa.at[pl.ds(cb, _CH)])
        pltpu.sync_copy(bb, ob.at[pl.ds(cb, _CH)])


@functools.lru_cache(maxsize=1)
def _make_gather():
    return pl.kernel(
        _gather_body,
        out_type=(
            jax.ShapeDtypeStruct((_B, _D), jnp.float32),
            jax.ShapeDtypeStruct((_B, _F), jnp.float32),
        ),
        mesh=plsc.VectorSubcoreMesh(core_axis_name="c", subcore_axis_name="s",
                                    num_cores=_NC, num_subcores=_NS),
        scratch_types=[
            pltpu.VMEM((_BPW,), jnp.int32),
            pltpu.VMEM((_CH, _D), jnp.float32),
            pltpu.VMEM((_CH, _F), jnp.float32),
            pltpu.SemaphoreType.DMA,
        ],
    )


def _tower(e, f, w1a, w1b, b1, g, beta, w2, b2):
    h = (jnp.dot(e, w1a, preferred_element_type=jnp.float32)
         + jnp.dot(f, w1b, preferred_element_type=jnp.float32) + b1)
    h = jnp.maximum(h, 0.0)
    m = jnp.mean(h, axis=-1, keepdims=True)
    v = jnp.mean(jnp.square(h - m), axis=-1, keepdims=True)
    h = (h - m) / jnp.sqrt(v + 1e-5) * g + beta
    z = jnp.dot(h, w2, preferred_element_type=jnp.float32) + b2
    n = jnp.sqrt(jnp.sum(z * z, axis=-1, keepdims=True))
    return z / jnp.maximum(n, 1e-12)


def _tower_body(ue, uf, ie, it,
                uw1a, uw1b, ub1, ug, ubeta, uw2, ub2,
                iw1a, iw1b, ib1, ig, ibeta, iw2, ib2, out):
    uv = _tower(ue[...], uf[...], uw1a[...], uw1b[...], ub1[...], ug[...],
                ubeta[...], uw2[...], ub2[...])
    iv = _tower(ie[...], it[...], iw1a[...], iw1b[...], ib1[...], ig[...],
                ibeta[...], iw2[...], ib2[...])
    out[...] = jnp.sum(uv * iv, axis=-1, keepdims=True)


def _row_spec(cols):
    return pl.BlockSpec((_BLK, cols), lambda i: (i, 0))


def _full_spec(r, c):
    return pl.BlockSpec((r, c), lambda i: (0, 0))


_towers = pl.pallas_call(
    _tower_body,
    grid=(_NBLK,),
    in_specs=[
        _row_spec(_D), _row_spec(_F), _row_spec(_D), _row_spec(_F),
        _full_spec(_D, _H), _full_spec(_F, _H), _full_spec(1, _H),
        _full_spec(1, _H), _full_spec(1, _H), _full_spec(_H, _D),
        _full_spec(1, _D),
        _full_spec(_D, _H), _full_spec(_F, _H), _full_spec(1, _H),
        _full_spec(1, _H), _full_spec(1, _H), _full_spec(_H, _D),
        _full_spec(1, _D),
    ],
    out_specs=pl.BlockSpec((_BLK, 1), lambda i: (i, 0)),
    out_shape=jax.ShapeDtypeStruct((_B, 1), jnp.float32),
)


def kernel(user_ids, item_ids, user_feats, item_feats, user_emb, item_emb,
           u_W1, u_b1, u_g, u_beta, u_W2, u_b2,
           i_W1, i_b1, i_g, i_beta, i_W2, i_b2):
    uids = user_ids.astype(jnp.int32)
    iids = item_ids.astype(jnp.int32)
    gie, git = _make_gather()(iids, item_emb, item_feats)
    gue, guf = _make_gather()(uids, user_emb, user_feats)
    scores = _towers(
        gue, guf, gie, git,
        u_W1[:_D], u_W1[_D:], u_b1.reshape(1, _H), u_g.reshape(1, _H),
        u_beta.reshape(1, _H), u_W2, u_b2.reshape(1, _D),
        i_W1[:_D], i_W1[_D:], i_b1.reshape(1, _H), i_g.reshape(1, _H),
        i_beta.reshape(1, _H), i_W2, i_b2.reshape(1, _D),
    )
    return scores.reshape(_B)
